# Initial kernel scaffold; baseline (speedup 1.0000x reference)
import jax
import jax.numpy as jnp
from jax.experimental import pallas as pl


def kernel(user_table, item_table, W_self, W_neigh, W_ih, W_hh, b_ih, b_hh, W_pred, ui_edges, purchase_edges, users_inds):
    raise NotImplementedError("write your pallas kernel here")


# speedup vs baseline: 2.1983x; 2.1983x over previous
import dataclasses
import functools

import jax
import jax.numpy as jnp
from jax import lax
from jax.experimental import pallas as pl
from jax.experimental.pallas import tpu as pltpu
from jax.experimental.pallas import tpu_sc as plsc

NC = 2
NS = 16
NW = NC * NS
BLK = 64
CH = 8

D = 128
NPRED = 12
NEG = -1e30


def _sc_compiler_params():
    cp = pltpu.CompilerParams()
    if "needs_layout_passes" in pltpu.CompilerParams.__dataclass_fields__:
        cp = dataclasses.replace(cp, needs_layout_passes=False)
    return cp


def _pad_edges(src, dst, n_rows, nb):
    e = src.shape[0]
    epad = NW * nb * BLK
    src = jnp.concatenate([src.astype(jnp.int32), jnp.zeros((epad - e,), jnp.int32)])
    dst = jnp.concatenate(
        [dst.astype(jnp.int32), jnp.full((epad - e,), n_rows, jnp.int32)])
    return src.reshape(NW, nb, BLK), dst.reshape(NW, nb, BLK)


def _zero_rows(rows_v):
    @pl.loop(0, BLK)
    def _(r):
        @pl.loop(0, D // 16)
        def _(c):
            rows_v[r, pl.ds(c * 16, 16)] = jnp.zeros((16,), jnp.float32)


def _zero_hist(hist_v, nr):
    @pl.loop(0, nr // 16)
    def _(i):
        hist_v[pl.ds(i * 16, 16)] = jnp.zeros((16,), jnp.float32)


def _accumulate(table_h, src_h, dst_h, widx, nb,
                src_c, dst_c, rows_v, hist_v, acc_sh):
    ones16 = jnp.ones((16,), jnp.float32)

    @pl.loop(0, nb // CH)
    def _(ch):
        pltpu.sync_copy(src_h.at[widx].at[pl.ds(ch * CH, CH)], src_c)
        pltpu.sync_copy(dst_h.at[widx].at[pl.ds(ch * CH, CH)], dst_c)

        @pl.loop(0, CH)
        def _(j):
            pltpu.sync_copy(table_h.at[src_c.at[j]], rows_v)
            pltpu.sync_copy(rows_v, acc_sh.at[dst_c.at[j]], add=True)

            @pl.loop(0, BLK // 16)
            def _(q):
                plsc.addupdate_scatter(
                    hist_v, [dst_c[j, pl.ds(q * 16, 16)]], ones16)


def _reduce_hist(hist_v, hist_sh, red_v, row0, rps, sid):
    pltpu.sync_copy(hist_v, hist_sh.at[sid])
    plsc.subcore_barrier()
    _zero_hist(hist_v, hist_v.shape[0])

    @pl.loop(0, NS)
    def _(s2):
        pltpu.sync_copy(hist_sh.at[s2].at[pl.ds(row0, rps)], red_v)

        @pl.loop(0, rps // 16)
        def _(i):
            hist_v[pl.ds(row0 + i * 16, 16)] = (
                hist_v[pl.ds(row0 + i * 16, 16)] + red_v[pl.ds(i * 16, 16)])


def _sc_segment_sum(table, src_w, dst_w, nr, nb):
    rps = nr // NS
    mesh = plsc.VectorSubcoreMesh(core_axis_name="c", subcore_axis_name="s")

    @functools.partial(
        pl.kernel,
        out_type=(
            jax.ShapeDtypeStruct((NC, nr, D), jnp.float32),
            jax.ShapeDtypeStruct((NC, nr), jnp.float32),
        ),
        mesh=mesh,
        compiler_params=_sc_compiler_params(),
        scratch_types=[
            pltpu.VMEM((CH, BLK), jnp.int32),
            pltpu.VMEM((CH, BLK), jnp.int32),
            pltpu.VMEM((BLK, D), jnp.float32),
            pltpu.VMEM((nr,), jnp.float32),
            pltpu.VMEM((nr // NS,), jnp.float32),
            pltpu.VMEM_SHARED((nr, D), jnp.float32),
            pltpu.VMEM_SHARED((NS, nr), jnp.float32),
        ],
    )
    def k(table_h, src_h, dst_h, acc_h, deg_h,
          src_c, dst_c, rows_v, hist_v, red_v, acc_sh, hist_sh):
        cid = lax.axis_index("c")
        sid = lax.axis_index("s")
        wid = cid * NS + sid
        row0 = sid * rps

        _zero_rows(rows_v)
        _zero_hist(hist_v, nr)

        @pl.loop(0, rps // BLK)
        def _(j):
            pltpu.sync_copy(rows_v, acc_sh.at[pl.ds(row0 + j * BLK, BLK)])
        plsc.subcore_barrier()

        _accumulate(table_h, src_h, dst_h, wid, nb,
                    src_c, dst_c, rows_v, hist_v, acc_sh)
        plsc.subcore_barrier()

        _reduce_hist(hist_v, hist_sh, red_v, row0, rps, sid)
        pltpu.sync_copy(hist_v.at[pl.ds(row0, rps)],
                        deg_h.at[cid].at[pl.ds(row0, rps)])
        pltpu.sync_copy(acc_sh.at[pl.ds(row0, rps)],
                        acc_h.at[cid].at[pl.ds(row0, rps)])

    return k(table, src_w, dst_w)


def _sc_purchase(item_embs, src_tw, dst_tw, users_sel, nr, nb, t_steps, nsel):
    sps = nsel // NS
    assert sps == BLK
    rps = nr // NS
    mesh = plsc.VectorSubcoreMesh(core_axis_name="c", subcore_axis_name="s")

    @functools.partial(
        pl.kernel,
        out_type=(
            jax.ShapeDtypeStruct((t_steps, NC, nsel, D), jnp.float32),
            jax.ShapeDtypeStruct((t_steps, NC, nsel), jnp.float32),
        ),
        mesh=mesh,
        compiler_params=_sc_compiler_params(),
        scratch_types=[
            pltpu.VMEM((CH, BLK), jnp.int32),
            pltpu.VMEM((CH, BLK), jnp.int32),
            pltpu.VMEM((BLK, D), jnp.float32),
            pltpu.VMEM((nr,), jnp.float32),
            pltpu.VMEM((nr // NS,), jnp.float32),
            pltpu.VMEM((sps,), jnp.int32),
            pltpu.VMEM((sps,), jnp.float32),
            pltpu.VMEM_SHARED((nr, D), jnp.float32),
            pltpu.VMEM_SHARED((NS, nr), jnp.float32),
            pltpu.VMEM_SHARED((nr,), jnp.float32),
        ],
    )
    def k(table_h, src_h, dst_h, sel_h, acc_h, deg_h,
          src_c, dst_c, rows_v, hist_v, red_v, sel_v, sdeg_v,
          acc_sh, hist_sh, deg_sh):
        cid = lax.axis_index("c")
        sid = lax.axis_index("s")
        wid = cid * NS + sid
        row0 = sid * rps

        pltpu.sync_copy(sel_h.at[sid], sel_v)

        @pl.loop(0, t_steps)
        def _(t):
            _zero_rows(rows_v)
            _zero_hist(hist_v, nr)

            @pl.loop(0, rps // BLK)
            def _(j):
                pltpu.sync_copy(rows_v, acc_sh.at[pl.ds(row0 + j * BLK, BLK)])
            plsc.subcore_barrier()

            _accumulate(table_h, src_h, dst_h, t * NW + wid, nb,
                        src_c, dst_c, rows_v, hist_v, acc_sh)
            plsc.subcore_barrier()

            _reduce_hist(hist_v, hist_sh, red_v, row0, rps, sid)
            pltpu.sync_copy(hist_v.at[pl.ds(row0, rps)],
                            deg_sh.at[pl.ds(row0, rps)])
            plsc.subcore_barrier()

            pltpu.sync_copy(acc_sh.at[sel_v], rows_v)
            pltpu.sync_copy(rows_v,
                            acc_h.at[t].at[cid].at[pl.ds(sid * sps, sps)])

            pltpu.sync_copy(deg_sh, hist_v)

            @pl.loop(0, sps // 16)
            def _(q):
                ids = sel_v[pl.ds(q * 16, 16)]
                sdeg_v[pl.ds(q * 16, 16)] = plsc.load_gather(hist_v, [ids])

            pltpu.sync_copy(sdeg_v,
                            deg_h.at[t].at[cid].at[pl.ds(sid * sps, sps)])
            plsc.subcore_barrier()

    return k(item_embs, src_tw, dst_tw, users_sel)


def _tc_item_embs(acc, deg, item_table, w_self, w_neigh):
    n = item_table.shape[0]
    blk = 1000

    def body(acc_r, deg_r, it_r, ws_r, wn_r, out_r):
        a = acc_r[0] + acc_r[1]
        d = deg_r[0] + deg_r[1]
        neigh = a / jnp.maximum(d, 1.0)
        out_r[...] = jax.nn.relu(
            jnp.dot(it_r[...], ws_r[...]) + jnp.dot(neigh, wn_r[...]))

    return pl.pallas_call(
        body,
        grid=(n // blk,),
        in_specs=[
            pl.BlockSpec((NC, blk, D), lambda i: (0, i, 0)),
            pl.BlockSpec((NC, blk, 1), lambda i: (0, i, 0)),
            pl.BlockSpec((blk, D), lambda i: (i, 0)),
            pl.BlockSpec((D, D), lambda i: (0, 0)),
            pl.BlockSpec((D, D), lambda i: (0, 0)),
        ],
        out_specs=pl.BlockSpec((blk, D), lambda i: (i, 0)),
        out_shape=jax.ShapeDtypeStruct((n, D), jnp.float32),
    )(acc, deg, item_table, w_self, w_neigh)


def _tc_gru(pacc, pdeg, w_ih, w_hh, b_ih, b_hh, w_pred, t_steps, nsel, h_dim):
    dt = (((1,), (1,)), ((), ()))

    def body(pacc_r, pdeg_r, wi_r, wh_r, bi_r, bh_r, wp_r, out_r):
        h = jnp.zeros((nsel, h_dim), jnp.float32)
        wi = wi_r[...]
        wh = wh_r[...]
        bi = bi_r[...]
        bh = bh_r[...]
        for t in range(t_steps):
            a = pacc_r[t, 0] + pacc_r[t, 1]
            d = pdeg_r[t, 0] + pdeg_r[t, 1]
            x = a / jnp.maximum(d, 1.0)
            gi = jax.lax.dot_general(x, wi, dt) + bi
            gh = jax.lax.dot_general(h, wh, dt) + bh
            i_r, i_z, i_n = gi[:, :h_dim], gi[:, h_dim:2 * h_dim], gi[:, 2 * h_dim:]
            h_r, h_z, h_n = gh[:, :h_dim], gh[:, h_dim:2 * h_dim], gh[:, 2 * h_dim:]
            r = jax.nn.sigmoid(i_r + h_r)
            z = jax.nn.sigmoid(i_z + h_z)
            nn = jnp.tanh(i_n + r * h_n)
            h = (1.0 - z) * nn + z * h
        out_r[...] = jnp.dot(h, wp_r[...])

    return pl.pallas_call(
        body,
        out_shape=jax.ShapeDtypeStruct((nsel, D), jnp.float32),
    )(pacc, pdeg, w_ih, w_hh, b_ih, b_hh, w_pred)


def _tc_topk(u, item_embs_pad, n_items, nsel, k_top):
    npad = item_embs_pad.shape[0]
    rblk = 128

    def body(u_r, ie_r, vals_r, idx_r):
        logits = jax.lax.dot_general(
            u_r[...], ie_r[...], (((1,), (1,)), ((), ())))
        cols = jax.lax.broadcasted_iota(jnp.int32, (rblk, npad), 1)
        l = jnp.where(cols < n_items, logits, NEG)
        vs = []
        ids = []
        for _ in range(k_top):
            m = jnp.max(l, axis=1)
            a = jnp.min(jnp.where(l == m[:, None], cols, npad),
                        axis=1).astype(jnp.int32)
            vs.append(m)
            ids.append(a)
            l = jnp.where(cols == a[:, None], NEG, l)
        vals_r[...] = jax.nn.sigmoid(jnp.stack(vs, axis=1))
        idx_r[...] = jnp.stack(ids, axis=1)

    return pl.pallas_call(
        body,
        grid=(nsel // rblk,),
        in_specs=[
            pl.BlockSpec((rblk, D), lambda i: (i, 0)),
            pl.BlockSpec((npad, D), lambda i: (0, 0)),
        ],
        out_specs=[
            pl.BlockSpec((rblk, k_top), lambda i: (i, 0)),
            pl.BlockSpec((rblk, k_top), lambda i: (i, 0)),
        ],
        out_shape=[
            jax.ShapeDtypeStruct((nsel, k_top), jnp.float32),
            jax.ShapeDtypeStruct((nsel, k_top), jnp.int32),
        ],
    )(u, item_embs_pad)


def kernel(user_table, item_table, W_self, W_neigh, W_ih, W_hh, b_ih, b_hh,
           W_pred, ui_edges, purchase_edges, users_inds):
    num_users, d = user_table.shape
    num_items = item_table.shape[0]
    h_dim = W_hh.shape[1]
    t_steps = purchase_edges.shape[0]
    e_gnn = ui_edges.shape[1]
    e_p = purchase_edges.shape[2]
    nsel = users_inds.shape[0]

    nr_items = ((num_items + 1 + NS * BLK - 1) // (NS * BLK)) * (NS * BLK)
    nr_users = ((num_users + 1 + NS * BLK - 1) // (NS * BLK)) * (NS * BLK)

    nb_gnn = ((e_gnn + NW * BLK - 1) // (NW * BLK) + CH - 1) // CH * CH
    nb_p = ((e_p + NW * BLK - 1) // (NW * BLK) + CH - 1) // CH * CH

    src_w, dst_w = _pad_edges(ui_edges[0], ui_edges[1], num_items, nb_gnn)

    ps = []
    pd = []
    for t in range(t_steps):
        s_w, d_w = _pad_edges(purchase_edges[t, 0], purchase_edges[t, 1],
                              num_users, nb_p)
        ps.append(s_w)
        pd.append(d_w)
    src_tw = jnp.concatenate(ps, axis=0)
    dst_tw = jnp.concatenate(pd, axis=0)
    users_sel = users_inds.astype(jnp.int32).reshape(NS, nsel // NS)

    acc, deg = _sc_segment_sum(user_table, src_w, dst_w, nr_items, nb_gnn)

    item_embs = _tc_item_embs(acc[:, :num_items], deg[:, :num_items, None],
                              item_table, W_self, W_neigh)

    pacc, pdeg = _sc_purchase(item_embs, src_tw, dst_tw, users_sel,
                              nr_users, nb_p, t_steps, nsel)

    u = _tc_gru(pacc, pdeg[..., None], W_ih, W_hh, b_ih.reshape(1, -1),
                b_hh.reshape(1, -1), W_pred, t_steps, nsel, h_dim)

    npad = ((num_items + 1023) // 1024) * 1024
    ie_pad = jnp.pad(item_embs, ((0, npad - num_items), (0, 0)))
    top_vals, top_idx = _tc_topk(u, ie_pad, num_items, nsel, NPRED)
    return top_vals, top_idx
